# baseline (speedup 1.0000x reference)
import functools

import jax
import jax.numpy as jnp
from jax.experimental import pallas as pl
from jax.experimental.pallas import tpu as pltpu

_CONTRACT_LAST = (((1,), (1,)), ((), ()))


def _fused_kernel(d_ref, w1_ref, b1_ref, w2_ref, b2_ref, w3_ref, b3_ref,
                  x_ref, o_ref, g_ref, bcol_ref, *, B, C, RC):
    i = pl.program_id(0)

    @pl.when(i == 0)
    def _():
        h = jax.lax.dot_general(w1_ref[...], d_ref[...], _CONTRACT_LAST,
                                preferred_element_type=jnp.float32)
        h = jnp.maximum(h + b1_ref[...], 0.0)
        h = jnp.dot(w2_ref[...], h, preferred_element_type=jnp.float32)
        h = jnp.maximum(h + b2_ref[...], 0.0)
        for b in range(B):
            col = jnp.dot(w3_ref[...], h[:, b:b + 1],
                          preferred_element_type=jnp.float32) + b3_ref[...]
            g_ref[b * C:(b + 1) * C, :] = col[0:C, :] + 1.0
            bcol_ref[b * C:(b + 1) * C, :] = col[C:2 * C, :]

    g = g_ref[pl.ds(i * RC, RC), :]
    bb = bcol_ref[pl.ds(i * RC, RC), :]
    o_ref[...] = g * x_ref[...] + bb


def kernel(x, d, w1, b1, w2, b2, w3, b3):
    B, C, H, W = x.shape
    HW = H * W
    x_flat = x.reshape(B * C, HW)
    b1r = b1.reshape(-1, 1)
    b2r = b2.reshape(-1, 1)
    b3r = b3.reshape(-1, 1)

    R = 16
    RC = R * C

    def whole(shape):
        n = len(shape)
        return pl.BlockSpec(shape, lambda i, _n=n: (0,) * _n)

    out = pl.pallas_call(
        functools.partial(_fused_kernel, B=B, C=C, RC=RC),
        out_shape=jax.ShapeDtypeStruct((B * C, HW), jnp.float32),
        grid=(B // R,),
        in_specs=[whole(d.shape), whole(w1.shape), whole(b1r.shape),
                  whole(w2.shape), whole(b2r.shape),
                  whole(w3.shape), whole(b3r.shape),
                  pl.BlockSpec((RC, HW), lambda i: (i, 0))],
        out_specs=pl.BlockSpec((RC, HW), lambda i: (i, 0)),
        scratch_shapes=[pltpu.VMEM((B * C, 1), jnp.float32),
                        pltpu.VMEM((B * C, 1), jnp.float32)],
        compiler_params=pltpu.CompilerParams(
            dimension_semantics=("arbitrary",),
            vmem_limit_bytes=44 << 20),
    )(d, w1, b1r, w2, b2r, w3, b3r, x_flat)

    return out.reshape(B, C, H, W)


# speedup vs baseline: 1.0435x; 1.0435x over previous
import functools

import jax
import jax.numpy as jnp
from jax.experimental import pallas as pl
from jax.experimental.pallas import tpu as pltpu

_CONTRACT_LAST = (((1,), (1,)), ((), ()))

_N_CHUNKS = 8
_DEPTH = 4


def _fused_kernel(d_ref, w1_ref, b1_ref, w2_ref, b2_ref, w3_ref, b3_ref,
                  x_any, o_any, ibuf, obuf, g_ref, bcol_ref, isem, osem,
                  *, B, C, rows):
    def cin(k):
        s = k % _DEPTH
        return pltpu.make_async_copy(
            x_any.at[pl.ds(k * rows, rows), :], ibuf.at[s], isem.at[s])

    def cout(k):
        s = k % _DEPTH
        return pltpu.make_async_copy(
            obuf.at[s], o_any.at[pl.ds(k * rows, rows), :], osem.at[s])

    for k in range(_DEPTH):
        cin(k).start()

    h = jax.lax.dot_general(w1_ref[...], d_ref[...], _CONTRACT_LAST,
                            preferred_element_type=jnp.float32)
    h = jnp.maximum(h + b1_ref[...], 0.0)
    h = jnp.dot(w2_ref[...], h, preferred_element_type=jnp.float32)
    h = jnp.maximum(h + b2_ref[...], 0.0)
    for b in range(B):
        col = jnp.dot(w3_ref[...], h[:, b:b + 1],
                      preferred_element_type=jnp.float32) + b3_ref[...]
        g_ref[b * C:(b + 1) * C, :] = col[0:C, :] + 1.0
        bcol_ref[b * C:(b + 1) * C, :] = col[C:2 * C, :]

    for k in range(_N_CHUNKS):
        s = k % _DEPTH
        cin(k).wait()
        if k >= _DEPTH:
            cout(k - _DEPTH).wait()
        g = g_ref[k * rows:(k + 1) * rows, :]
        bb = bcol_ref[k * rows:(k + 1) * rows, :]
        obuf[s, :, :] = g * ibuf[s, :, :] + bb
        cout(k).start()
        if k + _DEPTH < _N_CHUNKS:
            cin(k + _DEPTH).start()

    for k in range(_N_CHUNKS - _DEPTH, _N_CHUNKS):
        cout(k).wait()


def kernel(x, d, w1, b1, w2, b2, w3, b3):
    B, C, H, W = x.shape
    HW = H * W
    BC = B * C
    x_flat = x.reshape(BC, HW)
    b1r = b1.reshape(-1, 1)
    b2r = b2.reshape(-1, 1)
    b3r = b3.reshape(-1, 1)
    rows = BC // _N_CHUNKS

    def whole(shape):
        return pl.BlockSpec(shape, lambda *_: (0,) * len(shape))

    out = pl.pallas_call(
        functools.partial(_fused_kernel, B=B, C=C, rows=rows),
        out_shape=jax.ShapeDtypeStruct((BC, HW), jnp.float32),
        in_specs=[whole(d.shape), whole(w1.shape), whole(b1r.shape),
                  whole(w2.shape), whole(b2r.shape),
                  whole(w3.shape), whole(b3r.shape),
                  pl.BlockSpec(memory_space=pl.ANY)],
        out_specs=pl.BlockSpec(memory_space=pl.ANY),
        scratch_shapes=[pltpu.VMEM((_DEPTH, rows, HW), jnp.float32),
                        pltpu.VMEM((_DEPTH, rows, HW), jnp.float32),
                        pltpu.VMEM((BC, 1), jnp.float32),
                        pltpu.VMEM((BC, 1), jnp.float32),
                        pltpu.SemaphoreType.DMA((_DEPTH,)),
                        pltpu.SemaphoreType.DMA((_DEPTH,))],
        compiler_params=pltpu.CompilerParams(
            dimension_semantics=(),
            vmem_limit_bytes=44 << 20),
    )(d, w1, b1r, w2, b2r, w3, b3r, x_flat)

    return out.reshape(B, C, H, W)


# speedup vs baseline: 1.0541x; 1.0102x over previous
import functools

import jax
import jax.numpy as jnp
from jax.experimental import pallas as pl
from jax.experimental.pallas import tpu as pltpu

_CONTRACT_LAST = (((1,), (1,)), ((), ()))

_N_CHUNKS = 4
_DEPTH = 4


def _fused_kernel(d_ref, w1_ref, b1_ref, w2_ref, b2_ref, w3_ref, b3_ref,
                  x_any, o_any, ibuf, obuf, g_ref, bcol_ref, isem, osem,
                  *, B, C, rows):
    def cin(k):
        s = k % _DEPTH
        return pltpu.make_async_copy(
            x_any.at[pl.ds(k * rows, rows), :], ibuf.at[s], isem.at[s])

    def cout(k):
        s = k % _DEPTH
        return pltpu.make_async_copy(
            obuf.at[s], o_any.at[pl.ds(k * rows, rows), :], osem.at[s])

    for k in range(_DEPTH):
        cin(k).start()

    h = jax.lax.dot_general(w1_ref[...], d_ref[...], _CONTRACT_LAST,
                            preferred_element_type=jnp.float32)
    h = jnp.maximum(h + b1_ref[...], 0.0)
    h = jnp.dot(w2_ref[...], h, preferred_element_type=jnp.float32)
    h = jnp.maximum(h + b2_ref[...], 0.0)
    for b in range(B):
        col = jnp.dot(w3_ref[...], h[:, b:b + 1],
                      preferred_element_type=jnp.float32) + b3_ref[...]
        g_ref[b * C:(b + 1) * C, :] = col[0:C, :] + 1.0
        bcol_ref[b * C:(b + 1) * C, :] = col[C:2 * C, :]

    for k in range(_N_CHUNKS):
        s = k % _DEPTH
        cin(k).wait()
        if k >= _DEPTH:
            cout(k - _DEPTH).wait()
        g = g_ref[k * rows:(k + 1) * rows, :]
        bb = bcol_ref[k * rows:(k + 1) * rows, :]
        obuf[s, :, :] = g * ibuf[s, :, :] + bb
        cout(k).start()
        if k + _DEPTH < _N_CHUNKS:
            cin(k + _DEPTH).start()

    for k in range(_N_CHUNKS - _DEPTH, _N_CHUNKS):
        cout(k).wait()


def kernel(x, d, w1, b1, w2, b2, w3, b3):
    B, C, H, W = x.shape
    HW = H * W
    BC = B * C
    x_flat = x.reshape(BC, HW)
    b1r = b1.reshape(-1, 1)
    b2r = b2.reshape(-1, 1)
    b3r = b3.reshape(-1, 1)
    rows = BC // _N_CHUNKS

    def whole(shape):
        return pl.BlockSpec(shape, lambda *_: (0,) * len(shape))

    out = pl.pallas_call(
        functools.partial(_fused_kernel, B=B, C=C, rows=rows),
        out_shape=jax.ShapeDtypeStruct((BC, HW), jnp.float32),
        in_specs=[whole(d.shape), whole(w1.shape), whole(b1r.shape),
                  whole(w2.shape), whole(b2r.shape),
                  whole(w3.shape), whole(b3r.shape),
                  pl.BlockSpec(memory_space=pl.ANY)],
        out_specs=pl.BlockSpec(memory_space=pl.ANY),
        scratch_shapes=[pltpu.VMEM((_DEPTH, rows, HW), jnp.float32),
                        pltpu.VMEM((_DEPTH, rows, HW), jnp.float32),
                        pltpu.VMEM((BC, 1), jnp.float32),
                        pltpu.VMEM((BC, 1), jnp.float32),
                        pltpu.SemaphoreType.DMA((_DEPTH,)),
                        pltpu.SemaphoreType.DMA((_DEPTH,))],
        compiler_params=pltpu.CompilerParams(
            dimension_semantics=(),
            vmem_limit_bytes=44 << 20),
    )(d, w1, b1r, w2, b2r, w3, b3r, x_flat)

    return out.reshape(B, C, H, W)


# speedup vs baseline: 1.0619x; 1.0074x over previous
import functools

import jax
import jax.numpy as jnp
from jax.experimental import pallas as pl
from jax.experimental.pallas import tpu as pltpu

_CONTRACT_LAST = (((1,), (1,)), ((), ()))

_N_CHUNKS = 4
_DEPTH = 4


def _fused_kernel(d_any, w1_any, b1_any, w2_any, b2_any, w3_any, b3_any,
                  x_any, o_any, ibuf, obuf, g_ref, bcol_ref, isem, osem,
                  d_ref, w1_ref, b1_ref, w2_ref, b2_ref, w3_ref, b3_ref,
                  wsem, *, B, C, rows):
    def cin(k):
        s = k % _DEPTH
        return pltpu.make_async_copy(
            x_any.at[pl.ds(k * rows, rows), :], ibuf.at[s], isem.at[s])

    def cout(k):
        s = k % _DEPTH
        return pltpu.make_async_copy(
            obuf.at[s], o_any.at[pl.ds(k * rows, rows), :], osem.at[s])

    for k in range(_DEPTH):
        cin(k).start()

    wcopies = [pltpu.make_async_copy(src, dst, wsem.at[j]) for j, (src, dst)
               in enumerate([(d_any, d_ref), (w1_any, w1_ref),
                             (b1_any, b1_ref), (w2_any, w2_ref),
                             (b2_any, b2_ref), (w3_any, w3_ref),
                             (b3_any, b3_ref)])]
    for c in wcopies:
        c.start()
    for c in wcopies:
        c.wait()

    h = jax.lax.dot_general(w1_ref[...], d_ref[...], _CONTRACT_LAST,
                            preferred_element_type=jnp.float32)
    h = jnp.maximum(h + b1_ref[...], 0.0)
    h = jnp.dot(w2_ref[...], h, preferred_element_type=jnp.float32)
    h = jnp.maximum(h + b2_ref[...], 0.0)
    for b in range(B):
        col = jnp.dot(w3_ref[...], h[:, b:b + 1],
                      preferred_element_type=jnp.float32) + b3_ref[...]
        g_ref[b * C:(b + 1) * C, :] = col[0:C, :] + 1.0
        bcol_ref[b * C:(b + 1) * C, :] = col[C:2 * C, :]

    for k in range(_N_CHUNKS):
        s = k % _DEPTH
        cin(k).wait()
        if k >= _DEPTH:
            cout(k - _DEPTH).wait()
        g = g_ref[k * rows:(k + 1) * rows, :]
        bb = bcol_ref[k * rows:(k + 1) * rows, :]
        obuf[s, :, :] = g * ibuf[s, :, :] + bb
        cout(k).start()
        if k + _DEPTH < _N_CHUNKS:
            cin(k + _DEPTH).start()

    for k in range(_N_CHUNKS - _DEPTH, _N_CHUNKS):
        cout(k).wait()


def kernel(x, d, w1, b1, w2, b2, w3, b3):
    B, C, H, W = x.shape
    HW = H * W
    BC = B * C
    x_flat = x.reshape(BC, HW)
    b1r = b1.reshape(-1, 1)
    b2r = b2.reshape(-1, 1)
    b3r = b3.reshape(-1, 1)
    rows = BC // _N_CHUNKS

    anyspec = pl.BlockSpec(memory_space=pl.ANY)

    out = pl.pallas_call(
        functools.partial(_fused_kernel, B=B, C=C, rows=rows),
        out_shape=jax.ShapeDtypeStruct((BC, HW), jnp.float32),
        in_specs=[anyspec] * 8,
        out_specs=anyspec,
        scratch_shapes=[pltpu.VMEM((_DEPTH, rows, HW), jnp.float32),
                        pltpu.VMEM((_DEPTH, rows, HW), jnp.float32),
                        pltpu.VMEM((BC, 1), jnp.float32),
                        pltpu.VMEM((BC, 1), jnp.float32),
                        pltpu.SemaphoreType.DMA((_DEPTH,)),
                        pltpu.SemaphoreType.DMA((_DEPTH,)),
                        pltpu.VMEM(d.shape, jnp.float32),
                        pltpu.VMEM(w1.shape, jnp.float32),
                        pltpu.VMEM(b1r.shape, jnp.float32),
                        pltpu.VMEM(w2.shape, jnp.float32),
                        pltpu.VMEM(b2r.shape, jnp.float32),
                        pltpu.VMEM(w3.shape, jnp.float32),
                        pltpu.VMEM(b3r.shape, jnp.float32),
                        pltpu.SemaphoreType.DMA((7,))],
        compiler_params=pltpu.CompilerParams(
            dimension_semantics=(),
            vmem_limit_bytes=44 << 20),
    )(d, w1, b1r, w2, b2r, w3, b3r, x_flat)

    return out.reshape(B, C, H, W)


# speedup vs baseline: 1.0632x; 1.0013x over previous
import functools

import jax
import jax.numpy as jnp
from jax.experimental import pallas as pl
from jax.experimental.pallas import tpu as pltpu

_CONTRACT_LAST = (((1,), (1,)), ((), ()))

_N_CHUNKS = 4
_DEPTH = 4


def _fused_kernel(d_any, w1_any, b1_any, w2_any, b2_any, w3_any, b3_any,
                  x_any, o_any, ibuf, obuf, g_ref, bcol_ref, isem, osem,
                  d_ref, w1_ref, b1_ref, w2_ref, b2_ref, w3_ref, b3_ref,
                  wsem, *, B, C, rows):
    def cin(k):
        s = k % _DEPTH
        return pltpu.make_async_copy(
            x_any.at[pl.ds(k * rows, rows), :], ibuf.at[s], isem.at[s])

    def cout(k):
        s = k % _DEPTH
        return pltpu.make_async_copy(
            obuf.at[s], o_any.at[pl.ds(k * rows, rows), :], osem.at[s])

    for k in range(_DEPTH):
        cin(k).start()

    wcopies = [pltpu.make_async_copy(src, dst, wsem.at[j]) for j, (src, dst)
               in enumerate([(d_any, d_ref), (w1_any, w1_ref),
                             (b1_any, b1_ref), (w2_any, w2_ref),
                             (b2_any, b2_ref), (w3_any, w3_ref),
                             (b3_any, b3_ref)])]
    for c in wcopies:
        c.start()
    for c in wcopies:
        c.wait()

    h = jax.lax.dot_general(w1_ref[...], d_ref[...], _CONTRACT_LAST,
                            preferred_element_type=jnp.float32)
    h = jnp.maximum(h + b1_ref[...], 0.0)
    h = jnp.dot(w2_ref[...], h, preferred_element_type=jnp.float32)
    h = jnp.maximum(h + b2_ref[...], 0.0)
    for b in range(B):
        col = jnp.dot(w3_ref[...], h[:, b:b + 1],
                      preferred_element_type=jnp.float32) + b3_ref[...]
        g_ref[b * C:(b + 1) * C, :] = col[0:C, :] + 1.0
        bcol_ref[b * C:(b + 1) * C, :] = col[C:2 * C, :]

    for k in range(_N_CHUNKS):
        s = k % _DEPTH
        cin(k).wait()
        if k >= _DEPTH:
            cout(k - _DEPTH).wait()
        g = g_ref[k * rows:(k + 1) * rows, :]
        bb = bcol_ref[k * rows:(k + 1) * rows, :]
        obuf[s, :, :] = g * ibuf[s, :, :] + bb
        cout(k).start()
        if k + _DEPTH < _N_CHUNKS:
            cin(k + _DEPTH).start()

    for k in range(_N_CHUNKS - _DEPTH, _N_CHUNKS):
        cout(k).wait()


def kernel(x, d, w1, b1, w2, b2, w3, b3):
    B, C, H, W = x.shape
    HW = H * W
    BC = B * C
    x_flat = x.reshape(BC, HW)
    b1r = b1.reshape(-1, 1)
    b2r = b2.reshape(-1, 1)
    b3r = b3.reshape(-1, 1)
    rows = BC // _N_CHUNKS

    anyspec = pl.BlockSpec(memory_space=pl.ANY)

    out = pl.pallas_call(
        functools.partial(_fused_kernel, B=B, C=C, rows=rows),
        out_shape=jax.ShapeDtypeStruct((BC, HW), jnp.float32),
        in_specs=[anyspec] * 8,
        out_specs=anyspec,
        scratch_shapes=[pltpu.VMEM((_DEPTH, rows, HW), jnp.float32),
                        pltpu.VMEM((_DEPTH, rows, HW), jnp.float32),
                        pltpu.VMEM((BC, 1), jnp.float32),
                        pltpu.VMEM((BC, 1), jnp.float32),
                        pltpu.SemaphoreType.DMA((_DEPTH,)),
                        pltpu.SemaphoreType.DMA((_DEPTH,)),
                        pltpu.VMEM(d.shape, jnp.float32),
                        pltpu.VMEM(w1.shape, jnp.float32),
                        pltpu.VMEM(b1r.shape, jnp.float32),
                        pltpu.VMEM(w2.shape, jnp.float32),
                        pltpu.VMEM(b2r.shape, jnp.float32),
                        pltpu.VMEM(w3.shape, jnp.float32),
                        pltpu.VMEM(b3r.shape, jnp.float32),
                        pltpu.SemaphoreType.DMA((7,))],
        compiler_params=pltpu.CompilerParams(
            dimension_semantics=(),
            vmem_limit_bytes=44 << 20,
            disable_bounds_checks=True,
            disable_semaphore_checks=True),
    )(d, w1, b1r, w2, b2r, w3, b3r, x_flat)

    return out.reshape(B, C, H, W)


# speedup vs baseline: 1.0709x; 1.0072x over previous
import functools

import jax
import jax.numpy as jnp
from jax.experimental import pallas as pl
from jax.experimental.pallas import tpu as pltpu

_CONTRACT_LAST = (((1,), (1,)), ((), ()))

_N_CHUNKS = 8
_DEPTH = 8


def _fused_kernel(d_any, w1_any, b1_any, w2_any, b2_any, w3_any, b3_any,
                  x_any, o_any, ibuf, obuf, g_ref, bcol_ref, isem, osem,
                  d_ref, w1_ref, b1_ref, w2_ref, b2_ref, w3_ref, b3_ref,
                  wsem, *, B, C, rows):
    def cin(k):
        s = k % _DEPTH
        return pltpu.make_async_copy(
            x_any.at[pl.ds(k * rows, rows), :], ibuf.at[s], isem.at[s])

    def cout(k):
        s = k % _DEPTH
        return pltpu.make_async_copy(
            obuf.at[s], o_any.at[pl.ds(k * rows, rows), :], osem.at[s])

    for k in range(_DEPTH):
        cin(k).start()

    wcopies = [pltpu.make_async_copy(src, dst, wsem.at[j]) for j, (src, dst)
               in enumerate([(d_any, d_ref), (w1_any, w1_ref),
                             (b1_any, b1_ref), (w2_any, w2_ref),
                             (b2_any, b2_ref), (w3_any, w3_ref),
                             (b3_any, b3_ref)])]
    for c in wcopies:
        c.start()
    for c in wcopies:
        c.wait()

    h = jax.lax.dot_general(w1_ref[...], d_ref[...], _CONTRACT_LAST,
                            preferred_element_type=jnp.float32)
    h = jnp.maximum(h + b1_ref[...], 0.0)
    h = jnp.dot(w2_ref[...], h, preferred_element_type=jnp.float32)
    h = jnp.maximum(h + b2_ref[...], 0.0)
    for b in range(B):
        col = jnp.dot(w3_ref[...], h[:, b:b + 1],
                      preferred_element_type=jnp.float32) + b3_ref[...]
        g_ref[b * C:(b + 1) * C, :] = col[0:C, :] + 1.0
        bcol_ref[b * C:(b + 1) * C, :] = col[C:2 * C, :]

    for k in range(_N_CHUNKS):
        s = k % _DEPTH
        cin(k).wait()
        if k >= _DEPTH:
            cout(k - _DEPTH).wait()
        g = g_ref[k * rows:(k + 1) * rows, :]
        bb = bcol_ref[k * rows:(k + 1) * rows, :]
        obuf[s, :, :] = g * ibuf[s, :, :] + bb
        cout(k).start()
        if k + _DEPTH < _N_CHUNKS:
            cin(k + _DEPTH).start()

    for k in range(_N_CHUNKS - _DEPTH, _N_CHUNKS):
        cout(k).wait()


def kernel(x, d, w1, b1, w2, b2, w3, b3):
    B, C, H, W = x.shape
    HW = H * W
    BC = B * C
    x_flat = x.reshape(BC, HW)
    b1r = b1.reshape(-1, 1)
    b2r = b2.reshape(-1, 1)
    b3r = b3.reshape(-1, 1)
    rows = BC // _N_CHUNKS

    anyspec = pl.BlockSpec(memory_space=pl.ANY)

    out = pl.pallas_call(
        functools.partial(_fused_kernel, B=B, C=C, rows=rows),
        out_shape=jax.ShapeDtypeStruct((BC, HW), jnp.float32),
        in_specs=[anyspec] * 8,
        out_specs=anyspec,
        scratch_shapes=[pltpu.VMEM((_DEPTH, rows, HW), jnp.float32),
                        pltpu.VMEM((_DEPTH, rows, HW), jnp.float32),
                        pltpu.VMEM((BC, 1), jnp.float32),
                        pltpu.VMEM((BC, 1), jnp.float32),
                        pltpu.SemaphoreType.DMA((_DEPTH,)),
                        pltpu.SemaphoreType.DMA((_DEPTH,)),
                        pltpu.VMEM(d.shape, jnp.float32),
                        pltpu.VMEM(w1.shape, jnp.float32),
                        pltpu.VMEM(b1r.shape, jnp.float32),
                        pltpu.VMEM(w2.shape, jnp.float32),
                        pltpu.VMEM(b2r.shape, jnp.float32),
                        pltpu.VMEM(w3.shape, jnp.float32),
                        pltpu.VMEM(b3r.shape, jnp.float32),
                        pltpu.SemaphoreType.DMA((7,))],
        compiler_params=pltpu.CompilerParams(
            dimension_semantics=(),
            vmem_limit_bytes=44 << 20,
            disable_bounds_checks=True,
            disable_semaphore_checks=True),
    )(d, w1, b1r, w2, b2r, w3, b3r, x_flat)

    return out.reshape(B, C, H, W)


# speedup vs baseline: 1.0757x; 1.0045x over previous
import functools

import jax
import jax.numpy as jnp
from jax.experimental import pallas as pl
from jax.experimental.pallas import tpu as pltpu

_CONTRACT_LAST = (((1,), (1,)), ((), ()))

_N_CHUNKS = 16
_DEPTH = 16


def _fused_kernel(d_any, w1_any, b1_any, w2_any, b2_any, w3_any, b3_any,
                  x_any, o_any, ibuf, obuf, g_ref, bcol_ref, isem, osem,
                  d_ref, w1_ref, b1_ref, w2_ref, b2_ref, w3_ref, b3_ref,
                  wsem, *, B, C, rows):
    def cin(k):
        s = k % _DEPTH
        return pltpu.make_async_copy(
            x_any.at[pl.ds(k * rows, rows), :], ibuf.at[s], isem.at[s])

    def cout(k):
        s = k % _DEPTH
        return pltpu.make_async_copy(
            obuf.at[s], o_any.at[pl.ds(k * rows, rows), :], osem.at[s])

    for k in range(_DEPTH):
        cin(k).start()

    wcopies = [pltpu.make_async_copy(src, dst, wsem.at[j]) for j, (src, dst)
               in enumerate([(d_any, d_ref), (w1_any, w1_ref),
                             (b1_any, b1_ref), (w2_any, w2_ref),
                             (b2_any, b2_ref), (w3_any, w3_ref),
                             (b3_any, b3_ref)])]
    for c in wcopies:
        c.start()
    for c in wcopies:
        c.wait()

    h = jax.lax.dot_general(w1_ref[...], d_ref[...], _CONTRACT_LAST,
                            preferred_element_type=jnp.float32)
    h = jnp.maximum(h + b1_ref[...], 0.0)
    h = jnp.dot(w2_ref[...], h, preferred_element_type=jnp.float32)
    h = jnp.maximum(h + b2_ref[...], 0.0)
    for b in range(B):
        col = jnp.dot(w3_ref[...], h[:, b:b + 1],
                      preferred_element_type=jnp.float32) + b3_ref[...]
        g_ref[b * C:(b + 1) * C, :] = col[0:C, :] + 1.0
        bcol_ref[b * C:(b + 1) * C, :] = col[C:2 * C, :]

    for k in range(_N_CHUNKS):
        s = k % _DEPTH
        cin(k).wait()
        if k >= _DEPTH:
            cout(k - _DEPTH).wait()
        g = g_ref[k * rows:(k + 1) * rows, :]
        bb = bcol_ref[k * rows:(k + 1) * rows, :]
        obuf[s, :, :] = g * ibuf[s, :, :] + bb
        cout(k).start()
        if k + _DEPTH < _N_CHUNKS:
            cin(k + _DEPTH).start()

    for k in range(_N_CHUNKS - _DEPTH, _N_CHUNKS):
        cout(k).wait()


def kernel(x, d, w1, b1, w2, b2, w3, b3):
    B, C, H, W = x.shape
    HW = H * W
    BC = B * C
    x_flat = x.reshape(BC, HW)
    b1r = b1.reshape(-1, 1)
    b2r = b2.reshape(-1, 1)
    b3r = b3.reshape(-1, 1)
    rows = BC // _N_CHUNKS

    anyspec = pl.BlockSpec(memory_space=pl.ANY)

    out = pl.pallas_call(
        functools.partial(_fused_kernel, B=B, C=C, rows=rows),
        out_shape=jax.ShapeDtypeStruct((BC, HW), jnp.float32),
        in_specs=[anyspec] * 8,
        out_specs=anyspec,
        scratch_shapes=[pltpu.VMEM((_DEPTH, rows, HW), jnp.float32),
                        pltpu.VMEM((_DEPTH, rows, HW), jnp.float32),
                        pltpu.VMEM((BC, 1), jnp.float32),
                        pltpu.VMEM((BC, 1), jnp.float32),
                        pltpu.SemaphoreType.DMA((_DEPTH,)),
                        pltpu.SemaphoreType.DMA((_DEPTH,)),
                        pltpu.VMEM(d.shape, jnp.float32),
                        pltpu.VMEM(w1.shape, jnp.float32),
                        pltpu.VMEM(b1r.shape, jnp.float32),
                        pltpu.VMEM(w2.shape, jnp.float32),
                        pltpu.VMEM(b2r.shape, jnp.float32),
                        pltpu.VMEM(w3.shape, jnp.float32),
                        pltpu.VMEM(b3r.shape, jnp.float32),
                        pltpu.SemaphoreType.DMA((7,))],
        compiler_params=pltpu.CompilerParams(
            dimension_semantics=(),
            vmem_limit_bytes=44 << 20,
            disable_bounds_checks=True,
            disable_semaphore_checks=True),
    )(d, w1, b1r, w2, b2r, w3, b3r, x_flat)

    return out.reshape(B, C, H, W)
